# Initial kernel scaffold; baseline (speedup 1.0000x reference)
import jax
import jax.numpy as jnp
from jax import lax
from jax.experimental import pallas as pl
from jax.experimental.pallas import tpu as pltpu
from jax.experimental.pallas import tpu_sc as plsc

N = 10000
FM = 128
H = 8
E = 320000
OUT_CH = 128
N_CIRC = 504

NP = 10240
NC, NS = 2, 16
NW = NC * NS
E_REAL = E + N
TA = 10368
EP = TA * NW
KA = 128
KB = 64
ROWS_W = NP // NS

_MESH = plsc.VectorSubcoreMesh(core_axis_name="c", subcore_axis_name="s")


def _linear_body(xb, w_ref, asrc_ref, adst_ref, xw_ref, als_ref, ald_ref):
    xw = jnp.dot(xb, w_ref[...], preferred_element_type=jnp.float32)
    xw_ref[...] = xw
    row = lax.broadcasted_iota(jnp.int32, (H * FM, 16), 0) // FM
    col = lax.broadcasted_iota(jnp.int32, (H * FM, 16), 1)
    msk = (row == col).astype(jnp.float32)
    amat_s = jnp.reshape(asrc_ref[...], (H * FM, 1)) * msk
    amat_d = jnp.reshape(adst_ref[...], (H * FM, 1)) * msk
    als_ref[...] = jnp.dot(xw, amat_s, preferred_element_type=jnp.float32)
    ald_ref[...] = jnp.dot(xw, amat_d, preferred_element_type=jnp.float32)


def _linear1_kernel(x_ref, w_ref, asrc_ref, adst_ref, xw_ref, als_ref, ald_ref):
    _linear_body(x_ref[...], w_ref, asrc_ref, adst_ref, xw_ref, als_ref, ald_ref)


def _linear2_kernel(y0_ref, y1_ref, b_ref, w_ref, asrc_ref, adst_ref,
                    x_out_ref, xw_ref, als_ref, ald_ref):
    xb = jnp.maximum(y0_ref[...] + y1_ref[...] + b_ref[...], 0.0)
    x_out_ref[...] = xb
    _linear_body(xb, w_ref, asrc_ref, adst_ref, xw_ref, als_ref, ald_ref)


_BM = 512


def _tc_linear1(x_pad, W, a_src_f, a_dst_f):
    grid = NP // _BM
    return pl.pallas_call(
        _linear1_kernel,
        grid=(grid,),
        in_specs=[
            pl.BlockSpec((_BM, FM), lambda i: (i, 0)),
            pl.BlockSpec((FM, H * FM), lambda i: (0, 0)),
            pl.BlockSpec((1, H * FM), lambda i: (0, 0)),
            pl.BlockSpec((1, H * FM), lambda i: (0, 0)),
        ],
        out_specs=[
            pl.BlockSpec((_BM, H * FM), lambda i: (i, 0)),
            pl.BlockSpec((_BM, 16), lambda i: (i, 0)),
            pl.BlockSpec((_BM, 16), lambda i: (i, 0)),
        ],
        out_shape=[
            jax.ShapeDtypeStruct((NP, H * FM), jnp.float32),
            jax.ShapeDtypeStruct((NP, 16), jnp.float32),
            jax.ShapeDtypeStruct((NP, 16), jnp.float32),
        ],
    )(x_pad, W, a_src_f, a_dst_f)


def _tc_linear2(ypart, b, W, a_src_f, a_dst_f):
    grid = NP // _BM
    return pl.pallas_call(
        _linear2_kernel,
        grid=(grid,),
        in_specs=[
            pl.BlockSpec((_BM, FM), lambda i: (i, 0)),
            pl.BlockSpec((_BM, FM), lambda i: (i + NP // _BM, 0)),
            pl.BlockSpec((1, FM), lambda i: (0, 0)),
            pl.BlockSpec((FM, H * FM), lambda i: (0, 0)),
            pl.BlockSpec((1, H * FM), lambda i: (0, 0)),
            pl.BlockSpec((1, H * FM), lambda i: (0, 0)),
        ],
        out_specs=[
            pl.BlockSpec((_BM, FM), lambda i: (i, 0)),
            pl.BlockSpec((_BM, H * FM), lambda i: (i, 0)),
            pl.BlockSpec((_BM, 16), lambda i: (i, 0)),
            pl.BlockSpec((_BM, 16), lambda i: (i, 0)),
        ],
        out_shape=[
            jax.ShapeDtypeStruct((NP, FM), jnp.float32),
            jax.ShapeDtypeStruct((NP, H * FM), jnp.float32),
            jax.ShapeDtypeStruct((NP, 16), jnp.float32),
            jax.ShapeDtypeStruct((NP, 16), jnp.float32),
        ],
    )(ypart, ypart, b, W, a_src_f, a_dst_f)


def _dinv_kernel(d0_ref, d1_ref, out_ref):
    out_ref[...] = 1.0 / (H * (d0_ref[...] + d1_ref[...]) + H * 1e-16)


def _tc_dinv(denom_flat):
    blk = 1024
    return pl.pallas_call(
        _dinv_kernel,
        grid=(NP // blk,),
        in_specs=[
            pl.BlockSpec((blk, 16), lambda i: (i, 0)),
            pl.BlockSpec((blk, 16), lambda i: (i + NP // blk, 0)),
        ],
        out_specs=pl.BlockSpec((blk, 16), lambda i: (i, 0)),
        out_shape=jax.ShapeDtypeStruct((NP, 16), jnp.float32),
    )(denom_flat, denom_flat)


def _final_kernel(y0_ref, y1_ref, b2_ref, x1_ref, c1_ref, c2_ref, cb_ref, xo_ref):
    x2 = jnp.maximum(y0_ref[...] + y1_ref[...] + b2_ref[...], 0.0)
    xo_ref[...] = (
        jnp.dot(x1_ref[...], c1_ref[...], preferred_element_type=jnp.float32)
        + jnp.dot(x2, c2_ref[...], preferred_element_type=jnp.float32)
        + cb_ref[...]
    )


def _tc_final(ypart2, b2, x1, c1t, c2t, conv_b_row):
    grid = NP // _BM
    return pl.pallas_call(
        _final_kernel,
        grid=(grid,),
        in_specs=[
            pl.BlockSpec((_BM, FM), lambda i: (i, 0)),
            pl.BlockSpec((_BM, FM), lambda i: (i + NP // _BM, 0)),
            pl.BlockSpec((1, FM), lambda i: (0, 0)),
            pl.BlockSpec((_BM, FM), lambda i: (i, 0)),
            pl.BlockSpec((FM, OUT_CH), lambda i: (0, 0)),
            pl.BlockSpec((FM, OUT_CH), lambda i: (0, 0)),
            pl.BlockSpec((1, OUT_CH), lambda i: (0, 0)),
        ],
        out_specs=pl.BlockSpec((_BM, OUT_CH), lambda i: (i, 0)),
        out_shape=jax.ShapeDtypeStruct((NP, OUT_CH), jnp.float32),
    )(ypart2, ypart2, b2, x1, c1t, c2t, conv_b_row)


def _scores_kernel(a_ref, b_ref, out_ref):
    out_ref[...] = lax.dot_general(
        a_ref[...], b_ref[...], (((1,), (1,)), ((), ())),
        preferred_element_type=jnp.float32)


def _tc_scores(circ_pad, mirna_pad):
    bn = 1200
    nb = mirna_pad.shape[0] // bn
    return pl.pallas_call(
        _scores_kernel,
        grid=(nb,),
        in_specs=[
            pl.BlockSpec((512, FM), lambda i: (0, 0)),
            pl.BlockSpec((bn, FM), lambda i: (i, 0)),
        ],
        out_specs=pl.BlockSpec((512, bn), lambda i: (0, i)),
        out_shape=jax.ShapeDtypeStruct((512, nb * bn), jnp.float32),
    )(circ_pad, mirna_pad)


def _edge_att_body(src_hbm, dst_hbm, als_hbm, ald_hbm, zeros_hbm,
                   ex_out, denom_out,
                   sidx, didx, av, bv, exv, denom_sh, sem_a, sem_b):
    c = lax.axis_index("c")
    s = lax.axis_index("s")
    wid = s * NC + c
    pltpu.sync_copy(zeros_hbm, denom_sh.at[pl.ds(s * ROWS_W, ROWS_W)])
    plsc.subcore_barrier()

    base0 = wid * TA

    def batch(bi, _):
        base = base0 + bi * KA
        pltpu.sync_copy(src_hbm.at[pl.ds(base, KA)], sidx)
        pltpu.sync_copy(dst_hbm.at[pl.ds(base, KA)], didx)
        cp_a = pltpu.async_copy(als_hbm.at[sidx], av, sem_a)
        cp_b = pltpu.async_copy(ald_hbm.at[didx], bv, sem_b)
        cp_a.wait()
        cp_b.wait()

        def row(i, _):
            a = av[i, :] + bv[i, :]
            al = jnp.maximum(a, 0.2 * a)
            exv[i, :] = jnp.exp(al)
            return 0

        lax.fori_loop(0, KA, row, 0)
        pltpu.sync_copy(exv, ex_out.at[pl.ds(base, KA)])
        pltpu.sync_copy(exv, denom_sh.at[didx], add=True)
        return 0

    lax.fori_loop(0, TA // KA, batch, 0)
    plsc.subcore_barrier()
    pltpu.sync_copy(denom_sh.at[pl.ds(s * ROWS_W, ROWS_W)],
                    denom_out.at[pl.ds(c * NP + s * ROWS_W, ROWS_W)])


def _sc_edge_att(src_all, dst_all, als, ald, zeros16):
    k = pl.kernel(
        _edge_att_body,
        out_type=[
            jax.ShapeDtypeStruct((EP, 16), jnp.float32),
            jax.ShapeDtypeStruct((NC * NP, 16), jnp.float32),
        ],
        mesh=_MESH,
        scratch_types=[
            pltpu.VMEM((KA,), jnp.int32),
            pltpu.VMEM((KA,), jnp.int32),
            pltpu.VMEM((KA, 16), jnp.float32),
            pltpu.VMEM((KA, 16), jnp.float32),
            pltpu.VMEM((KA, 16), jnp.float32),
            pltpu.VMEM_SHARED((NP, 16), jnp.float32),
            pltpu.SemaphoreType.DMA,
            pltpu.SemaphoreType.DMA,
        ],
    )
    return k(src_all, dst_all, als, ald, zeros16)


def _edge_agg_body(src_hbm, dst_hbm, xw_hbm, ex_hbm, dinv_hbm, zeros_hbm,
                   y_out,
                   sidx, didx, xv, exv, dv, wv, cv, y_sh, sem_x, sem_d):
    c = lax.axis_index("c")
    s = lax.axis_index("s")
    wid = s * NC + c
    for z in range(ROWS_W // KB):
        pltpu.sync_copy(zeros_hbm, y_sh.at[pl.ds(s * ROWS_W + z * KB, KB)])
    plsc.subcore_barrier()

    base0 = wid * TA

    def batch(bi, _):
        base = base0 + bi * KB
        pltpu.sync_copy(src_hbm.at[pl.ds(base, KB)], sidx)
        pltpu.sync_copy(dst_hbm.at[pl.ds(base, KB)], didx)
        cp_x = pltpu.async_copy(xw_hbm.at[sidx], xv, sem_x)
        cp_d = pltpu.async_copy(dinv_hbm.at[didx], dv, sem_d)
        pltpu.sync_copy(ex_hbm.at[pl.ds(base, KB)], exv)
        cp_d.wait()
        cp_x.wait()

        def edge(i, _):
            wv[i, :] = exv[i, :] * dv[i, :]
            acc = [jnp.zeros((16,), jnp.float32) for _ in range(FM // 16)]
            for h in range(H):
                w = jnp.full((16,), wv[i, h])
                for j in range(FM // 16):
                    acc[j] = acc[j] + w * xv[i, pl.ds(h * FM + j * 16, 16)]
            for j in range(FM // 16):
                cv[i, pl.ds(j * 16, 16)] = acc[j]
            return 0

        lax.fori_loop(0, KB, edge, 0)
        pltpu.sync_copy(cv, y_sh.at[didx], add=True)
        return 0

    lax.fori_loop(0, TA // KB, batch, 0)
    plsc.subcore_barrier()
    pltpu.sync_copy(y_sh.at[pl.ds(s * ROWS_W, ROWS_W)],
                    y_out.at[pl.ds(c * NP + s * ROWS_W, ROWS_W)])


def _sc_edge_agg(src_all, dst_all, xw, ex, dinv, zeros128):
    k = pl.kernel(
        _edge_agg_body,
        out_type=jax.ShapeDtypeStruct((NC * NP, FM), jnp.float32),
        mesh=_MESH,
        scratch_types=[
            pltpu.VMEM((KB,), jnp.int32),
            pltpu.VMEM((KB,), jnp.int32),
            pltpu.VMEM((KB, H * FM), jnp.float32),
            pltpu.VMEM((KB, 16), jnp.float32),
            pltpu.VMEM((KB, 16), jnp.float32),
            pltpu.VMEM((KB, 16), jnp.float32),
            pltpu.VMEM((KB, FM), jnp.float32),
            pltpu.VMEM_SHARED((NP, FM), jnp.float32),
            pltpu.SemaphoreType.DMA,
            pltpu.SemaphoreType.DMA,
        ],
    )
    return k(src_all, dst_all, xw, ex, dinv, zeros128)


def _gat_layer(src_all, dst_all, xw, als, ald, z16, z128):
    ex, denom = _sc_edge_att(src_all, dst_all, als, ald, z16)
    dinv = _tc_dinv(denom)
    return _sc_edge_agg(src_all, dst_all, xw, ex, dinv, z128)


def kernel(x, edge_index, W1, a_src1, a_dst1, b1, W2, a_src2, a_dst2, b2,
           conv_w, conv_b):
    x_pad = jnp.zeros((NP, FM), jnp.float32).at[:N].set(x)
    loops = jnp.arange(N, dtype=jnp.int32)
    dummy = jnp.full((EP - E_REAL,), N, dtype=jnp.int32)
    src_all = jnp.concatenate([edge_index[0], loops, dummy])
    dst_all = jnp.concatenate([edge_index[1], loops, dummy])
    a_src1_f = a_src1.reshape(1, H * FM)
    a_dst1_f = a_dst1.reshape(1, H * FM)
    a_src2_f = a_src2.reshape(1, H * FM)
    a_dst2_f = a_dst2.reshape(1, H * FM)
    b1_row = b1.reshape(1, FM)
    b2_row = b2.reshape(1, FM)
    c1t = conv_w[:, 0, :, 0].T
    c2t = conv_w[:, 1, :, 0].T
    conv_b_row = conv_b.reshape(1, OUT_CH)
    z16 = jnp.zeros((ROWS_W, 16), jnp.float32)
    z128 = jnp.zeros((KB, FM), jnp.float32)

    xw1, als1, ald1 = _tc_linear1(x_pad, W1, a_src1_f, a_dst1_f)
    y1part = _gat_layer(src_all, dst_all, xw1, als1, ald1, z16, z128)

    x1, xw2, als2, ald2 = _tc_linear2(y1part, b1_row, W2, a_src2_f, a_dst2_f)
    y2part = _gat_layer(src_all, dst_all, xw2, als2, ald2, z16, z128)

    xo = _tc_final(y2part, b2_row, x1, c1t, c2t, conv_b_row)
    circ_pad = xo[:512]
    mirna_pad = xo[N_CIRC:N_CIRC + 9600]
    scores_full = _tc_scores(circ_pad, mirna_pad)

    circ = xo[:N_CIRC]
    mirna = xo[N_CIRC:N]
    scores = scores_full[:N_CIRC, :N - N_CIRC]
    return scores, circ, mirna


# speedup vs baseline: 19.7261x; 19.7261x over previous
import jax
import jax.numpy as jnp
from jax import lax
from jax.experimental import pallas as pl
from jax.experimental.pallas import tpu as pltpu
from jax.experimental.pallas import tpu_sc as plsc

N = 10000
FM = 128
H = 8
E = 320000
OUT_CH = 128
N_CIRC = 504

NP = 10240
NC, NS = 2, 16
NW = NC * NS
E_REAL = E + N
TA = 10368
EP = TA * NW
KA = 64
KB = 32
ZR = 32
ROWS_W = NP // NS

_MESH = plsc.VectorSubcoreMesh(core_axis_name="c", subcore_axis_name="s")


def _linear_body(xb, w_ref, asrc_ref, adst_ref, xw_ref, als_ref, ald_ref):
    xw = jnp.dot(xb, w_ref[...], preferred_element_type=jnp.float32)
    xw_ref[...] = xw
    row = lax.broadcasted_iota(jnp.int32, (H * FM, FM), 0) // FM
    col = lax.broadcasted_iota(jnp.int32, (H * FM, FM), 1)
    msk = (row == col).astype(jnp.float32)
    amat_s = jnp.reshape(asrc_ref[...], (H * FM, 1)) * msk
    amat_d = jnp.reshape(adst_ref[...], (H * FM, 1)) * msk
    als_ref[...] = jnp.dot(xw, amat_s, preferred_element_type=jnp.float32)
    ald_ref[...] = jnp.dot(xw, amat_d, preferred_element_type=jnp.float32)


def _linear1_kernel(x_ref, w_ref, asrc_ref, adst_ref, xw_ref, als_ref, ald_ref):
    _linear_body(x_ref[...], w_ref, asrc_ref, adst_ref, xw_ref, als_ref, ald_ref)


def _linear2_kernel(y0_ref, y1_ref, b_ref, w_ref, asrc_ref, adst_ref,
                    x_out_ref, xw_ref, als_ref, ald_ref):
    xb = jnp.maximum(y0_ref[...] + y1_ref[...] + b_ref[...], 0.0)
    x_out_ref[...] = xb
    _linear_body(xb, w_ref, asrc_ref, adst_ref, xw_ref, als_ref, ald_ref)


_BM = 512


def _tc_linear1(x_pad, W, a_src_f, a_dst_f):
    grid = NP // _BM
    return pl.pallas_call(
        _linear1_kernel,
        grid=(grid,),
        in_specs=[
            pl.BlockSpec((_BM, FM), lambda i: (i, 0)),
            pl.BlockSpec((FM, H * FM), lambda i: (0, 0)),
            pl.BlockSpec((1, H * FM), lambda i: (0, 0)),
            pl.BlockSpec((1, H * FM), lambda i: (0, 0)),
        ],
        out_specs=[
            pl.BlockSpec((_BM, H * FM), lambda i: (i, 0)),
            pl.BlockSpec((_BM, FM), lambda i: (i, 0)),
            pl.BlockSpec((_BM, FM), lambda i: (i, 0)),
        ],
        out_shape=[
            jax.ShapeDtypeStruct((NP, H * FM), jnp.float32),
            jax.ShapeDtypeStruct((NP, FM), jnp.float32),
            jax.ShapeDtypeStruct((NP, FM), jnp.float32),
        ],
    )(x_pad, W, a_src_f, a_dst_f)


def _tc_linear2(ypart, b, W, a_src_f, a_dst_f):
    grid = NP // _BM
    return pl.pallas_call(
        _linear2_kernel,
        grid=(grid,),
        in_specs=[
            pl.BlockSpec((_BM, FM), lambda i: (i, 0)),
            pl.BlockSpec((_BM, FM), lambda i: (i + NP // _BM, 0)),
            pl.BlockSpec((1, FM), lambda i: (0, 0)),
            pl.BlockSpec((FM, H * FM), lambda i: (0, 0)),
            pl.BlockSpec((1, H * FM), lambda i: (0, 0)),
            pl.BlockSpec((1, H * FM), lambda i: (0, 0)),
        ],
        out_specs=[
            pl.BlockSpec((_BM, FM), lambda i: (i, 0)),
            pl.BlockSpec((_BM, H * FM), lambda i: (i, 0)),
            pl.BlockSpec((_BM, FM), lambda i: (i, 0)),
            pl.BlockSpec((_BM, FM), lambda i: (i, 0)),
        ],
        out_shape=[
            jax.ShapeDtypeStruct((NP, FM), jnp.float32),
            jax.ShapeDtypeStruct((NP, H * FM), jnp.float32),
            jax.ShapeDtypeStruct((NP, FM), jnp.float32),
            jax.ShapeDtypeStruct((NP, FM), jnp.float32),
        ],
    )(ypart, ypart, b, W, a_src_f, a_dst_f)


def _dinv_kernel(d0_ref, d1_ref, out_ref):
    out_ref[...] = 1.0 / (H * (d0_ref[...] + d1_ref[...]) + H * 1e-16)


def _tc_dinv(denom_flat):
    blk = 1024
    return pl.pallas_call(
        _dinv_kernel,
        grid=(NP // blk,),
        in_specs=[
            pl.BlockSpec((blk, FM), lambda i: (i, 0)),
            pl.BlockSpec((blk, FM), lambda i: (i + NP // blk, 0)),
        ],
        out_specs=pl.BlockSpec((blk, FM), lambda i: (i, 0)),
        out_shape=jax.ShapeDtypeStruct((NP, FM), jnp.float32),
    )(denom_flat, denom_flat)


def _final_kernel(y0_ref, y1_ref, b2_ref, x1_ref, c1_ref, c2_ref, cb_ref, xo_ref):
    x2 = jnp.maximum(y0_ref[...] + y1_ref[...] + b2_ref[...], 0.0)
    xo_ref[...] = (
        jnp.dot(x1_ref[...], c1_ref[...], preferred_element_type=jnp.float32)
        + jnp.dot(x2, c2_ref[...], preferred_element_type=jnp.float32)
        + cb_ref[...]
    )


def _tc_final(ypart2, b2, x1, c1t, c2t, conv_b_row):
    grid = NP // _BM
    return pl.pallas_call(
        _final_kernel,
        grid=(grid,),
        in_specs=[
            pl.BlockSpec((_BM, FM), lambda i: (i, 0)),
            pl.BlockSpec((_BM, FM), lambda i: (i + NP // _BM, 0)),
            pl.BlockSpec((1, FM), lambda i: (0, 0)),
            pl.BlockSpec((_BM, FM), lambda i: (i, 0)),
            pl.BlockSpec((FM, OUT_CH), lambda i: (0, 0)),
            pl.BlockSpec((FM, OUT_CH), lambda i: (0, 0)),
            pl.BlockSpec((1, OUT_CH), lambda i: (0, 0)),
        ],
        out_specs=pl.BlockSpec((_BM, OUT_CH), lambda i: (i, 0)),
        out_shape=jax.ShapeDtypeStruct((NP, OUT_CH), jnp.float32),
    )(ypart2, ypart2, b2, x1, c1t, c2t, conv_b_row)


def _scores_kernel(a_ref, b_ref, out_ref):
    out_ref[...] = lax.dot_general(
        a_ref[...], b_ref[...], (((1,), (1,)), ((), ())),
        preferred_element_type=jnp.float32)


def _tc_scores(circ_pad, mirna_pad):
    bn = 640
    nb = mirna_pad.shape[0] // bn
    return pl.pallas_call(
        _scores_kernel,
        grid=(nb,),
        in_specs=[
            pl.BlockSpec((512, FM), lambda i: (0, 0)),
            pl.BlockSpec((bn, FM), lambda i: (i, 0)),
        ],
        out_specs=pl.BlockSpec((512, bn), lambda i: (0, i)),
        out_shape=jax.ShapeDtypeStruct((512, nb * bn), jnp.float32),
    )(circ_pad, mirna_pad)


def _zero_acc(acc_sh, zeros_hbm, zv, s):
    pltpu.sync_copy(zeros_hbm, zv)

    def z(i, _):
        pltpu.sync_copy(zv, acc_sh.at[pl.ds(s * ROWS_W + i * ZR, ZR)])
        return 0

    lax.fori_loop(0, ROWS_W // ZR, z, 0)


def _drain_acc(acc_sh, out_hbm, zv, c, s):
    def d(i, _):
        pltpu.sync_copy(acc_sh.at[pl.ds(s * ROWS_W + i * ZR, ZR)], zv)
        pltpu.sync_copy(zv, out_hbm.at[pl.ds(c * NP + s * ROWS_W + i * ZR, ZR)])
        return 0

    lax.fori_loop(0, ROWS_W // ZR, d, 0)


def _edge_att_body(src_hbm, dst_hbm, als_hbm, ald_hbm, zeros_hbm,
                   ex_out, denom_out,
                   sidx, didx, av, bv, exl, exs, zv, denom_sh, sem_a, sem_b):
    c = lax.axis_index("c")
    s = lax.axis_index("s")
    wid = s * NC + c
    _zero_acc(denom_sh, zeros_hbm, zv, s)
    pltpu.sync_copy(zeros_hbm, exs.at[pl.ds(0, ZR)])
    pltpu.sync_copy(zeros_hbm, exs.at[pl.ds(ZR, ZR)])
    plsc.subcore_barrier()

    base0 = wid * TA

    def batch(bi, _):
        base = base0 + bi * KA
        pltpu.sync_copy(src_hbm.at[pl.ds(base, KA)], sidx)
        pltpu.sync_copy(dst_hbm.at[pl.ds(base, KA)], didx)
        cp_a = pltpu.async_copy(als_hbm.at[sidx], av, sem_a)
        cp_b = pltpu.async_copy(ald_hbm.at[didx], bv, sem_b)
        cp_a.wait()
        cp_b.wait()

        def row(i, _):
            a = av[i, pl.ds(0, 16)] + bv[i, pl.ds(0, 16)]
            al = jnp.maximum(a, 0.2 * a)
            e = jnp.exp(al)
            exl[pl.ds(i * 16, 16)] = e
            exs[i, pl.ds(0, 16)] = e
            return 0

        lax.fori_loop(0, KA, row, 0)
        pltpu.sync_copy(exl, ex_out.at[pl.ds(base * 16, KA * 16)])
        pltpu.sync_copy(exs, denom_sh.at[didx], add=True)
        return 0

    lax.fori_loop(0, TA // KA, batch, 0)
    plsc.subcore_barrier()
    _drain_acc(denom_sh, denom_out, zv, c, s)


def _sc_edge_att(src_all, dst_all, als, ald, zeros):
    k = pl.kernel(
        _edge_att_body,
        out_type=[
            jax.ShapeDtypeStruct((EP * 16,), jnp.float32),
            jax.ShapeDtypeStruct((NC * NP, FM), jnp.float32),
        ],
        mesh=_MESH,
        scratch_types=[
            pltpu.VMEM((KA,), jnp.int32),
            pltpu.VMEM((KA,), jnp.int32),
            pltpu.VMEM((KA, FM), jnp.float32),
            pltpu.VMEM((KA, FM), jnp.float32),
            pltpu.VMEM((KA * 16,), jnp.float32),
            pltpu.VMEM((KA, FM), jnp.float32),
            pltpu.VMEM((ZR, FM), jnp.float32),
            pltpu.VMEM_SHARED((NP, FM), jnp.float32),
            pltpu.SemaphoreType.DMA,
            pltpu.SemaphoreType.DMA,
        ],
    )
    return k(src_all, dst_all, als, ald, zeros)


def _edge_agg_body(src_hbm, dst_hbm, xw_hbm, ex_hbm, dinv_hbm, zeros_hbm,
                   y_out,
                   sidx, didx, xv, exl, dv, cv, zv, y_sh, sem_x, sem_d):
    c = lax.axis_index("c")
    s = lax.axis_index("s")
    wid = s * NC + c
    _zero_acc(y_sh, zeros_hbm, zv, s)
    plsc.subcore_barrier()

    base0 = wid * TA

    def batch(bi, _):
        base = base0 + bi * KB
        pltpu.sync_copy(src_hbm.at[pl.ds(base, KB)], sidx)
        pltpu.sync_copy(dst_hbm.at[pl.ds(base, KB)], didx)
        cp_x = pltpu.async_copy(xw_hbm.at[sidx], xv, sem_x)
        cp_d = pltpu.async_copy(dinv_hbm.at[didx], dv, sem_d)
        pltpu.sync_copy(ex_hbm.at[pl.ds(base * 16, KB * 16)], exl)
        cp_d.wait()
        cp_x.wait()

        def edge(i, _):
            wrow = exl[pl.ds(i * 16, 16)] * dv[i, pl.ds(0, 16)]
            acc = [jnp.zeros((16,), jnp.float32) for _ in range(FM // 16)]
            for h in range(H):
                w = jnp.full((16,), wrow[h])
                for j in range(FM // 16):
                    acc[j] = acc[j] + w * xv[i, pl.ds(h * FM + j * 16, 16)]
            for j in range(FM // 16):
                cv[i, pl.ds(j * 16, 16)] = acc[j]
            return 0

        lax.fori_loop(0, KB, edge, 0)
        pltpu.sync_copy(cv, y_sh.at[didx], add=True)
        return 0

    lax.fori_loop(0, TA // KB, batch, 0)
    plsc.subcore_barrier()
    _drain_acc(y_sh, y_out, zv, c, s)


def _sc_edge_agg(src_all, dst_all, xw, ex, dinv, zeros):
    k = pl.kernel(
        _edge_agg_body,
        out_type=jax.ShapeDtypeStruct((NC * NP, FM), jnp.float32),
        mesh=_MESH,
        scratch_types=[
            pltpu.VMEM((KB,), jnp.int32),
            pltpu.VMEM((KB,), jnp.int32),
            pltpu.VMEM((KB, H * FM), jnp.float32),
            pltpu.VMEM((KB * 16,), jnp.float32),
            pltpu.VMEM((KB, FM), jnp.float32),
            pltpu.VMEM((KB, FM), jnp.float32),
            pltpu.VMEM((ZR, FM), jnp.float32),
            pltpu.VMEM_SHARED((NP, FM), jnp.float32),
            pltpu.SemaphoreType.DMA,
            pltpu.SemaphoreType.DMA,
        ],
    )
    return k(src_all, dst_all, xw, ex, dinv, zeros)


def _gat_layer(src_all, dst_all, xw, als, ald, zeros):
    ex, denom = _sc_edge_att(src_all, dst_all, als, ald, zeros)
    dinv = _tc_dinv(denom)
    return _sc_edge_agg(src_all, dst_all, xw, ex, dinv, zeros)


def kernel(x, edge_index, W1, a_src1, a_dst1, b1, W2, a_src2, a_dst2, b2,
           conv_w, conv_b):
    x_pad = jnp.zeros((NP, FM), jnp.float32).at[:N].set(x)
    loops = jnp.arange(N, dtype=jnp.int32)
    dummy = jnp.full((EP - E_REAL,), N, dtype=jnp.int32)
    src_all = jnp.concatenate([edge_index[0], loops, dummy])
    dst_all = jnp.concatenate([edge_index[1], loops, dummy])
    a_src1_f = a_src1.reshape(1, H * FM)
    a_dst1_f = a_dst1.reshape(1, H * FM)
    a_src2_f = a_src2.reshape(1, H * FM)
    a_dst2_f = a_dst2.reshape(1, H * FM)
    b1_row = b1.reshape(1, FM)
    b2_row = b2.reshape(1, FM)
    c1t = conv_w[:, 0, :, 0].T
    c2t = conv_w[:, 1, :, 0].T
    conv_b_row = conv_b.reshape(1, OUT_CH)
    zeros = jnp.zeros((ZR, FM), jnp.float32)

    xw1, als1, ald1 = _tc_linear1(x_pad, W1, a_src1_f, a_dst1_f)
    y1part = _gat_layer(src_all, dst_all, xw1, als1, ald1, zeros)

    x1, xw2, als2, ald2 = _tc_linear2(y1part, b1_row, W2, a_src2_f, a_dst2_f)
    y2part = _gat_layer(src_all, dst_all, xw2, als2, ald2, zeros)

    xo = _tc_final(y2part, b2_row, x1, c1t, c2t, conv_b_row)
    circ_pad = xo[:512]
    mirna_pad = xo[N_CIRC:N_CIRC + 9600]
    scores_full = _tc_scores(circ_pad, mirna_pad)

    circ = xo[:N_CIRC]
    mirna = xo[N_CIRC:N]
    scores = scores_full[:N_CIRC, :N - N_CIRC]
    return scores, circ, mirna


# speedup vs baseline: 20.1823x; 1.0231x over previous
import jax
import jax.numpy as jnp
from jax import lax
from jax.experimental import pallas as pl
from jax.experimental.pallas import tpu as pltpu
from jax.experimental.pallas import tpu_sc as plsc

N = 10000
FM = 128
H = 8
E = 320000
OUT_CH = 128
N_CIRC = 504

NP = 10240
NC, NS = 2, 16
NW = NC * NS
E_REAL = E + N
TA = 10368
EP = TA * NW
KA = 64
KB = 16
ZR = 32
ROWS_W = NP // NS

_MESH = plsc.VectorSubcoreMesh(core_axis_name="c", subcore_axis_name="s")


def _linear_body(xb, w_ref, asrc_ref, adst_ref, xw_ref, als_ref, ald_ref):
    xw = jnp.dot(xb, w_ref[...], preferred_element_type=jnp.float32)
    xw_ref[...] = xw
    row = lax.broadcasted_iota(jnp.int32, (H * FM, FM), 0) // FM
    col = lax.broadcasted_iota(jnp.int32, (H * FM, FM), 1)
    msk = (row == col).astype(jnp.float32)
    amat_s = jnp.reshape(asrc_ref[...], (H * FM, 1)) * msk
    amat_d = jnp.reshape(adst_ref[...], (H * FM, 1)) * msk
    als_ref[...] = jnp.dot(xw, amat_s, preferred_element_type=jnp.float32)
    ald_ref[...] = jnp.dot(xw, amat_d, preferred_element_type=jnp.float32)


def _linear1_kernel(x_ref, w_ref, asrc_ref, adst_ref, xw_ref, als_ref, ald_ref):
    _linear_body(x_ref[...], w_ref, asrc_ref, adst_ref, xw_ref, als_ref, ald_ref)


def _linear2_kernel(y0_ref, y1_ref, b_ref, w_ref, asrc_ref, adst_ref,
                    x_out_ref, xw_ref, als_ref, ald_ref):
    xb = jnp.maximum(y0_ref[...] + y1_ref[...] + b_ref[...], 0.0)
    x_out_ref[...] = xb
    _linear_body(xb, w_ref, asrc_ref, adst_ref, xw_ref, als_ref, ald_ref)


_BM = 512


def _tc_linear1(x_pad, W, a_src_f, a_dst_f):
    grid = NP // _BM
    return pl.pallas_call(
        _linear1_kernel,
        grid=(grid,),
        in_specs=[
            pl.BlockSpec((_BM, FM), lambda i: (i, 0)),
            pl.BlockSpec((FM, H * FM), lambda i: (0, 0)),
            pl.BlockSpec((1, H * FM), lambda i: (0, 0)),
            pl.BlockSpec((1, H * FM), lambda i: (0, 0)),
        ],
        out_specs=[
            pl.BlockSpec((_BM, H * FM), lambda i: (i, 0)),
            pl.BlockSpec((_BM, FM), lambda i: (i, 0)),
            pl.BlockSpec((_BM, FM), lambda i: (i, 0)),
        ],
        out_shape=[
            jax.ShapeDtypeStruct((NP, H * FM), jnp.float32),
            jax.ShapeDtypeStruct((NP, FM), jnp.float32),
            jax.ShapeDtypeStruct((NP, FM), jnp.float32),
        ],
    )(x_pad, W, a_src_f, a_dst_f)


def _tc_linear2(ypart, b, W, a_src_f, a_dst_f):
    grid = NP // _BM
    return pl.pallas_call(
        _linear2_kernel,
        grid=(grid,),
        in_specs=[
            pl.BlockSpec((_BM, FM), lambda i: (i, 0)),
            pl.BlockSpec((_BM, FM), lambda i: (i + NP // _BM, 0)),
            pl.BlockSpec((1, FM), lambda i: (0, 0)),
            pl.BlockSpec((FM, H * FM), lambda i: (0, 0)),
            pl.BlockSpec((1, H * FM), lambda i: (0, 0)),
            pl.BlockSpec((1, H * FM), lambda i: (0, 0)),
        ],
        out_specs=[
            pl.BlockSpec((_BM, FM), lambda i: (i, 0)),
            pl.BlockSpec((_BM, H * FM), lambda i: (i, 0)),
            pl.BlockSpec((_BM, FM), lambda i: (i, 0)),
            pl.BlockSpec((_BM, FM), lambda i: (i, 0)),
        ],
        out_shape=[
            jax.ShapeDtypeStruct((NP, FM), jnp.float32),
            jax.ShapeDtypeStruct((NP, H * FM), jnp.float32),
            jax.ShapeDtypeStruct((NP, FM), jnp.float32),
            jax.ShapeDtypeStruct((NP, FM), jnp.float32),
        ],
    )(ypart, ypart, b, W, a_src_f, a_dst_f)


def _dinv_kernel(d0_ref, d1_ref, out_ref):
    out_ref[...] = 1.0 / (H * (d0_ref[...] + d1_ref[...]) + H * 1e-16)


def _tc_dinv(denom_flat):
    blk = 1024
    return pl.pallas_call(
        _dinv_kernel,
        grid=(NP // blk,),
        in_specs=[
            pl.BlockSpec((blk, FM), lambda i: (i, 0)),
            pl.BlockSpec((blk, FM), lambda i: (i + NP // blk, 0)),
        ],
        out_specs=pl.BlockSpec((blk, FM), lambda i: (i, 0)),
        out_shape=jax.ShapeDtypeStruct((NP, FM), jnp.float32),
    )(denom_flat, denom_flat)


def _final_kernel(y0_ref, y1_ref, b2_ref, x1_ref, c1_ref, c2_ref, cb_ref, xo_ref):
    x2 = jnp.maximum(y0_ref[...] + y1_ref[...] + b2_ref[...], 0.0)
    xo_ref[...] = (
        jnp.dot(x1_ref[...], c1_ref[...], preferred_element_type=jnp.float32)
        + jnp.dot(x2, c2_ref[...], preferred_element_type=jnp.float32)
        + cb_ref[...]
    )


def _tc_final(ypart2, b2, x1, c1t, c2t, conv_b_row):
    grid = NP // _BM
    return pl.pallas_call(
        _final_kernel,
        grid=(grid,),
        in_specs=[
            pl.BlockSpec((_BM, FM), lambda i: (i, 0)),
            pl.BlockSpec((_BM, FM), lambda i: (i + NP // _BM, 0)),
            pl.BlockSpec((1, FM), lambda i: (0, 0)),
            pl.BlockSpec((_BM, FM), lambda i: (i, 0)),
            pl.BlockSpec((FM, OUT_CH), lambda i: (0, 0)),
            pl.BlockSpec((FM, OUT_CH), lambda i: (0, 0)),
            pl.BlockSpec((1, OUT_CH), lambda i: (0, 0)),
        ],
        out_specs=pl.BlockSpec((_BM, OUT_CH), lambda i: (i, 0)),
        out_shape=jax.ShapeDtypeStruct((NP, OUT_CH), jnp.float32),
    )(ypart2, ypart2, b2, x1, c1t, c2t, conv_b_row)


def _scores_kernel(a_ref, b_ref, out_ref):
    out_ref[...] = lax.dot_general(
        a_ref[...], b_ref[...], (((1,), (1,)), ((), ())),
        preferred_element_type=jnp.float32)


def _tc_scores(circ_pad, mirna_pad):
    bn = 640
    nb = mirna_pad.shape[0] // bn
    return pl.pallas_call(
        _scores_kernel,
        grid=(nb,),
        in_specs=[
            pl.BlockSpec((512, FM), lambda i: (0, 0)),
            pl.BlockSpec((bn, FM), lambda i: (i, 0)),
        ],
        out_specs=pl.BlockSpec((512, bn), lambda i: (0, i)),
        out_shape=jax.ShapeDtypeStruct((512, nb * bn), jnp.float32),
    )(circ_pad, mirna_pad)


def _zero_acc(acc_sh, zeros_hbm, zv, s):
    pltpu.sync_copy(zeros_hbm, zv)

    def z(i, _):
        pltpu.sync_copy(zv, acc_sh.at[pl.ds(s * ROWS_W + i * ZR, ZR)])
        return 0

    lax.fori_loop(0, ROWS_W // ZR, z, 0)


def _drain_acc(acc_sh, out_hbm, zv, c, s):
    def d(i, _):
        pltpu.sync_copy(acc_sh.at[pl.ds(s * ROWS_W + i * ZR, ZR)], zv)
        pltpu.sync_copy(zv, out_hbm.at[pl.ds(c * NP + s * ROWS_W + i * ZR, ZR)])
        return 0

    lax.fori_loop(0, ROWS_W // ZR, d, 0)


def _edge_att_body(src_hbm, dst_hbm, als_hbm, ald_hbm, zeros_hbm,
                   ex_out, denom_out,
                   sidx, didx, av, bv, exl, exs, zv, denom_sh, sem_a, sem_b):
    c = lax.axis_index("c")
    s = lax.axis_index("s")
    wid = s * NC + c
    _zero_acc(denom_sh, zeros_hbm, zv, s)
    pltpu.sync_copy(zeros_hbm, exs.at[pl.ds(0, ZR)])
    pltpu.sync_copy(zeros_hbm, exs.at[pl.ds(ZR, ZR)])
    plsc.subcore_barrier()

    base0 = wid * TA

    def batch(bi, _):
        base = base0 + bi * KA
        pltpu.sync_copy(src_hbm.at[pl.ds(base, KA)], sidx)
        pltpu.sync_copy(dst_hbm.at[pl.ds(base, KA)], didx)
        cp_a = pltpu.async_copy(als_hbm.at[sidx], av, sem_a)
        cp_b = pltpu.async_copy(ald_hbm.at[didx], bv, sem_b)
        cp_a.wait()
        cp_b.wait()

        def row(i, _):
            a = av[i, pl.ds(0, 16)] + bv[i, pl.ds(0, 16)]
            al = jnp.maximum(a, 0.2 * a)
            e = jnp.exp(al)
            exl[pl.ds(i * 16, 16)] = e
            exs[i, pl.ds(0, 16)] = e
            return 0

        lax.fori_loop(0, KA, row, 0)
        pltpu.sync_copy(exl, ex_out.at[pl.ds(base * 16, KA * 16)])
        pltpu.sync_copy(exs, denom_sh.at[didx], add=True)
        return 0

    lax.fori_loop(0, TA // KA, batch, 0)
    plsc.subcore_barrier()
    _drain_acc(denom_sh, denom_out, zv, c, s)


def _sc_edge_att(src_all, dst_all, als, ald, zeros):
    k = pl.kernel(
        _edge_att_body,
        out_type=[
            jax.ShapeDtypeStruct((EP * 16,), jnp.float32),
            jax.ShapeDtypeStruct((NC * NP, FM), jnp.float32),
        ],
        mesh=_MESH,
        scratch_types=[
            pltpu.VMEM((KA,), jnp.int32),
            pltpu.VMEM((KA,), jnp.int32),
            pltpu.VMEM((KA, FM), jnp.float32),
            pltpu.VMEM((KA, FM), jnp.float32),
            pltpu.VMEM((KA * 16,), jnp.float32),
            pltpu.VMEM((KA, FM), jnp.float32),
            pltpu.VMEM((ZR, FM), jnp.float32),
            pltpu.VMEM_SHARED((NP, FM), jnp.float32),
            pltpu.SemaphoreType.DMA,
            pltpu.SemaphoreType.DMA,
        ],
    )
    return k(src_all, dst_all, als, ald, zeros)


def _edge_agg_body(src_hbm, dst_hbm, xw_hbm, ex_hbm, dinv_hbm, zeros_hbm,
                   y_out,
                   sidx0, sidx1, didx0, didx1, xv0, xv1, exl0, exl1,
                   dv0, dv1, cv, zv, y_sh,
                   sem_x0, sem_x1, sem_d0, sem_d1, sem_e0, sem_e1):
    c = lax.axis_index("c")
    s = lax.axis_index("s")
    wid = s * NC + c
    _zero_acc(y_sh, zeros_hbm, zv, s)
    plsc.subcore_barrier()

    base0 = wid * TA
    nb = TA // KB
    bufs = ((sidx0, didx0, xv0, dv0, exl0, sem_x0, sem_d0, sem_e0),
            (sidx1, didx1, xv1, dv1, exl1, sem_x1, sem_d1, sem_e1))

    def issue(bi, b):
        si, di, xv, dv, exl, sx, sd, se = bufs[b]
        base = base0 + bi * KB
        pltpu.sync_copy(src_hbm.at[pl.ds(base, KB)], si)
        pltpu.sync_copy(dst_hbm.at[pl.ds(base, KB)], di)
        pltpu.async_copy(xw_hbm.at[si], xv, sx)
        pltpu.async_copy(dinv_hbm.at[di], dv, sd)
        pltpu.async_copy(ex_hbm.at[pl.ds(base * 16, KB * 16)], exl, se)

    def step(bi, b):
        si, di, xv, dv, exl, sx, sd, se = bufs[b]
        pltpu.make_async_copy(xw_hbm.at[si], xv, sx).wait()
        pltpu.make_async_copy(dinv_hbm.at[di], dv, sd).wait()
        pltpu.make_async_copy(ex_hbm.at[pl.ds(0, KB * 16)], exl, se).wait()

        @pl.when(bi + 1 < nb)
        def _():
            issue(bi + 1, 1 - b)

        def edge(i, _):
            wrow = exl[pl.ds(i * 16, 16)] * dv[i, pl.ds(0, 16)]
            acc = [jnp.zeros((16,), jnp.float32) for _ in range(FM // 16)]
            for h in range(H):
                w = jnp.full((16,), wrow[h])
                for j in range(FM // 16):
                    acc[j] = acc[j] + w * xv[i, pl.ds(h * FM + j * 16, 16)]
            for j in range(FM // 16):
                cv[i, pl.ds(j * 16, 16)] = acc[j]
            return 0

        lax.fori_loop(0, KB, edge, 0)
        pltpu.sync_copy(cv, y_sh.at[di], add=True)

    issue(0, 0)

    def pair(g, _):
        for b in range(2):
            step(2 * g + b, b)
        return 0

    lax.fori_loop(0, nb // 2, pair, 0)
    plsc.subcore_barrier()
    _drain_acc(y_sh, y_out, zv, c, s)


def _sc_edge_agg(src_all, dst_all, xw, ex, dinv, zeros):
    k = pl.kernel(
        _edge_agg_body,
        out_type=jax.ShapeDtypeStruct((NC * NP, FM), jnp.float32),
        mesh=_MESH,
        scratch_types=[
            pltpu.VMEM((KB,), jnp.int32),
            pltpu.VMEM((KB,), jnp.int32),
            pltpu.VMEM((KB,), jnp.int32),
            pltpu.VMEM((KB,), jnp.int32),
            pltpu.VMEM((KB, H * FM), jnp.float32),
            pltpu.VMEM((KB, H * FM), jnp.float32),
            pltpu.VMEM((KB * 16,), jnp.float32),
            pltpu.VMEM((KB * 16,), jnp.float32),
            pltpu.VMEM((KB, FM), jnp.float32),
            pltpu.VMEM((KB, FM), jnp.float32),
            pltpu.VMEM((KB, FM), jnp.float32),
            pltpu.VMEM((ZR, FM), jnp.float32),
            pltpu.VMEM_SHARED((NP, FM), jnp.float32),
            pltpu.SemaphoreType.DMA,
            pltpu.SemaphoreType.DMA,
            pltpu.SemaphoreType.DMA,
            pltpu.SemaphoreType.DMA,
            pltpu.SemaphoreType.DMA,
            pltpu.SemaphoreType.DMA,
        ],
    )
    return k(src_all, dst_all, xw, ex, dinv, zeros)


def _gat_layer(src_all, dst_all, xw, als, ald, zeros):
    ex, denom = _sc_edge_att(src_all, dst_all, als, ald, zeros)
    dinv = _tc_dinv(denom)
    return _sc_edge_agg(src_all, dst_all, xw, ex, dinv, zeros)


def kernel(x, edge_index, W1, a_src1, a_dst1, b1, W2, a_src2, a_dst2, b2,
           conv_w, conv_b):
    x_pad = jnp.zeros((NP, FM), jnp.float32).at[:N].set(x)
    loops = jnp.arange(N, dtype=jnp.int32)
    dummy = jnp.full((EP - E_REAL,), N, dtype=jnp.int32)
    src_all = jnp.concatenate([edge_index[0], loops, dummy])
    dst_all = jnp.concatenate([edge_index[1], loops, dummy])
    a_src1_f = a_src1.reshape(1, H * FM)
    a_dst1_f = a_dst1.reshape(1, H * FM)
    a_src2_f = a_src2.reshape(1, H * FM)
    a_dst2_f = a_dst2.reshape(1, H * FM)
    b1_row = b1.reshape(1, FM)
    b2_row = b2.reshape(1, FM)
    c1t = conv_w[:, 0, :, 0].T
    c2t = conv_w[:, 1, :, 0].T
    conv_b_row = conv_b.reshape(1, OUT_CH)
    zeros = jnp.zeros((ZR, FM), jnp.float32)

    xw1, als1, ald1 = _tc_linear1(x_pad, W1, a_src1_f, a_dst1_f)
    y1part = _gat_layer(src_all, dst_all, xw1, als1, ald1, zeros)

    x1, xw2, als2, ald2 = _tc_linear2(y1part, b1_row, W2, a_src2_f, a_dst2_f)
    y2part = _gat_layer(src_all, dst_all, xw2, als2, ald2, zeros)

    xo = _tc_final(y2part, b2_row, x1, c1t, c2t, conv_b_row)
    circ_pad = xo[:512]
    mirna_pad = xo[N_CIRC:N_CIRC + 9600]
    scores_full = _tc_scores(circ_pad, mirna_pad)

    circ = xo[:N_CIRC]
    mirna = xo[N_CIRC:N]
    scores = scores_full[:N_CIRC, :N - N_CIRC]
    return scores, circ, mirna
